# baseline (speedup 1.0000x reference)
import jax
import jax.numpy as jnp
from jax import lax
from jax.experimental import pallas as pl
from jax.experimental.pallas import tpu as pltpu
from jax.experimental.pallas import tpu_sc as plsc

import numpy as np

B, L, S = 1024, 50, 8
HIDDEN = 64
N_ITEMS = 100000

NC, NS = 2, 16
NW = NC * NS
IDX_PER_W = (B * L * S) // NW
G = 128
RG = G // S
NG = IDX_PER_W // G
ROWS_PER_W = IDX_PER_W // S
NB = 10
HU = HIDDEN // 2

F32_ROWS = N_ITEMS * HIDDEN // 128
CH = 128
BASE_RPW = F32_ROWS // NW
EXTRA_W = (F32_ROWS - BASE_RPW * NW) // 2
NCH = 13
PK_ROWS = F32_ROWS // 2

_RND = np.uint32(0x8000)
_HI = np.uint32(0xFFFF0000)


def _conv_body(tf32_hbm, tb_hbm, ibuf, obuf, isems, osems):
    wid = lax.axis_index("s") * NC + lax.axis_index("c")
    r0 = wid * BASE_RPW + 2 * jnp.minimum(wid, EXTRA_W)
    n = BASE_RPW + jnp.where(wid < EXTRA_W, 2, 0)

    def start_of(c):
        return r0 + jnp.minimum(c * CH, n - CH)

    def in_copy(c, bb):
        return pltpu.make_async_copy(
            tf32_hbm.at[pl.ds(start_of(c), CH)], ibuf.at[bb], isems[bb])

    def out_copy(c, bb):
        return pltpu.make_async_copy(
            obuf.at[bb], tb_hbm.at[pl.ds(start_of(c) // 2, CH // 2)],
            osems[bb])

    in_copy(0, 0).start()
    in_copy(1, 1).start()

    def do_chunk(cc, bb):
        in_copy(cc, bb).wait()

        @pl.when(cc >= 2)
        def _():
            out_copy(cc - 2, bb).wait()

        def row(rp, carry):
            for parity in range(2):
                rr = 2 * rp + parity
                for g in range(4):
                    ua = plsc.bitcast(ibuf[bb, rr, pl.ds(g * 32, 16)],
                                      jnp.uint32)
                    ub = plsc.bitcast(ibuf[bb, rr, pl.ds(g * 32 + 16, 16)],
                                      jnp.uint32)
                    w = ((ua + _RND) >> 16) | ((ub + _RND) & _HI)
                    obuf[bb, rp, pl.ds(parity * 64 + g * 16, 16)] = w
            return carry

        lax.fori_loop(0, CH // 2, row, 0, unroll=2)
        out_copy(cc, bb).start()

        @pl.when(cc + 2 < NCH)
        def _():
            in_copy(cc + 2, bb).start()

    def chunk(c, carry):
        do_chunk(c * 2, 0)
        do_chunk(c * 2 + 1, 1)
        return carry

    lax.fori_loop(0, NCH // 2, chunk, 0)
    do_chunk(NCH - 1, 0)
    out_copy(NCH - 2, 1).wait()
    out_copy(NCH - 1, 0).wait()


def _body(idx_hbm, table_hbm, out_hbm, idx_v, gbuf, obuf, osems, *sems):
    wid = lax.axis_index("s") * NC + lax.axis_index("c")
    out_base = wid * ROWS_PER_W

    pltpu.sync_copy(idx_hbm.at[pl.ds(wid * NG, NG)], idx_v)

    def gather(j, b):
        return pltpu.make_async_copy(
            table_hbm.at[idx_v.at[j]], gbuf.at[b], sems[b])

    def out_copy(j, ob):
        return pltpu.make_async_copy(
            obuf.at[ob], out_hbm.at[pl.ds(out_base + j * RG, RG)], osems[ob])

    for b in range(NB):
        gather(b, b).start()

    def group(g, carry):
        for b in range(NB):
            j = g * NB + b
            ob = b % 2
            gather(j, b).wait()

            @pl.when(j >= 2)
            def _():
                out_copy(j - 2, ob).wait()

            def row(r, carry):
                base = r * S

                def bf(s, h):
                    return plsc.bitcast(
                        gbuf[b, base + s, pl.ds(h * 16, 16)], jnp.bfloat16)

                for h in range(2):
                    t0 = bf(0, h) + bf(1, h)
                    t1 = bf(2, h) + bf(3, h)
                    t2 = bf(4, h) + bf(5, h)
                    t3 = bf(6, h) + bf(7, h)
                    u = plsc.bitcast((t0 + t1) + (t2 + t3), jnp.uint32)
                    lo = plsc.bitcast(u << 16, jnp.float32)
                    hi = plsc.bitcast(u & _HI, jnp.float32)
                    obuf[ob, r, pl.ds(h * 32, 16)] = lo * (1.0 / S)
                    obuf[ob, r, pl.ds(h * 32 + 16, 16)] = hi * (1.0 / S)
                return carry

            lax.fori_loop(0, RG, row, 0, unroll=4)
            out_copy(j, ob).start()

            @pl.when(j + NB < NG)
            def _():
                gather(j + NB, b).start()
        return carry

    lax.fori_loop(0, NG // NB, group, 0)
    out_copy(NG - 2, 0).wait()
    out_copy(NG - 1, 1).wait()


@jax.jit
def _run(idx, tf32):
    mesh = plsc.VectorSubcoreMesh(
        core_axis_name="c", subcore_axis_name="s",
        num_cores=NC, num_subcores=NS)
    conv = pl.kernel(
        _conv_body,
        out_type=jax.ShapeDtypeStruct((PK_ROWS, 128), jnp.uint32),
        mesh=mesh,
        scratch_types=[
            pltpu.VMEM((2, CH, 128), jnp.float32),
            pltpu.VMEM((2, CH // 2, 128), jnp.uint32),
            [pltpu.SemaphoreType.DMA] * 2,
            [pltpu.SemaphoreType.DMA] * 2,
        ],
        compiler_params=pltpu.CompilerParams(
            use_tc_tiling_on_sc=False, needs_layout_passes=False),
    )
    table_bf = conv(tf32).reshape(N_ITEMS, HU)
    kfn = pl.kernel(
        _body,
        out_type=jax.ShapeDtypeStruct((B * L, HIDDEN), jnp.float32),
        mesh=mesh,
        scratch_types=[
            pltpu.VMEM((NG, G), jnp.int32),
            pltpu.VMEM((NB, G, HU), jnp.uint32),
            pltpu.VMEM((2, RG, HIDDEN), jnp.float32),
            [pltpu.SemaphoreType.DMA] * 2,
        ] + [pltpu.SemaphoreType.DMA] * NB,
        compiler_params=pltpu.CompilerParams(
            use_tc_tiling_on_sc=False, needs_layout_passes=False),
    )
    return kfn(idx, table_bf)


def kernel(batch_basket, item_embedding):
    out = _run(batch_basket.reshape(NW * NG, G),
               item_embedding.reshape(F32_ROWS, 128))
    return out.reshape(B, L, HIDDEN)


# speedup vs baseline: 1.0019x; 1.0019x over previous
import jax
import jax.numpy as jnp
from jax import lax
from jax.experimental import pallas as pl
from jax.experimental.pallas import tpu as pltpu
from jax.experimental.pallas import tpu_sc as plsc

import numpy as np

B, L, S = 1024, 50, 8
HIDDEN = 64
N_ITEMS = 100000

NC, NS = 2, 16
NW = NC * NS
IDX_PER_W = (B * L * S) // NW
G = 128
RG = G // S
NG = IDX_PER_W // G
ROWS_PER_W = IDX_PER_W // S
NB = 10
HU = HIDDEN // 2

F32_ROWS = N_ITEMS * HIDDEN // 128
CH = 128
BASE_RPW = F32_ROWS // NW
EXTRA_W = (F32_ROWS - BASE_RPW * NW) // 2
NCH = 13
PK_ROWS = F32_ROWS // 2

_RND = np.uint32(0x8000)
_HI = np.uint32(0xFFFF0000)


def _conv_body(tf32_hbm, tb_hbm, ibuf, obuf, isems, osems):
    wid = lax.axis_index("s") * NC + lax.axis_index("c")
    r0 = wid * BASE_RPW + 2 * jnp.minimum(wid, EXTRA_W)
    n = BASE_RPW + jnp.where(wid < EXTRA_W, 2, 0)

    def start_of(c):
        return r0 + jnp.minimum(c * CH, n - CH)

    def in_copy(c, bb):
        return pltpu.make_async_copy(
            tf32_hbm.at[pl.ds(start_of(c), CH)], ibuf.at[bb], isems[bb])

    def out_copy(c, bb):
        return pltpu.make_async_copy(
            obuf.at[bb], tb_hbm.at[pl.ds(start_of(c) // 2, CH // 2)],
            osems[bb])

    in_copy(0, 0).start()
    in_copy(1, 1).start()

    def do_chunk(cc, bb):
        in_copy(cc, bb).wait()

        @pl.when(cc >= 2)
        def _():
            out_copy(cc - 2, bb).wait()

        def row(rp, carry):
            for parity in range(2):
                rr = 2 * rp + parity
                for g in range(4):
                    ua = plsc.bitcast(ibuf[bb, rr, pl.ds(g * 32, 16)],
                                      jnp.uint32)
                    ub = plsc.bitcast(ibuf[bb, rr, pl.ds(g * 32 + 16, 16)],
                                      jnp.uint32)
                    w = ((ua + _RND) >> 16) | ((ub + _RND) & _HI)
                    obuf[bb, rp, pl.ds(parity * 64 + g * 16, 16)] = w
            return carry

        lax.fori_loop(0, CH // 2, row, 0, unroll=2)
        out_copy(cc, bb).start()

        @pl.when(cc + 2 < NCH)
        def _():
            in_copy(cc + 2, bb).start()

    def chunk(c, carry):
        do_chunk(c * 2, 0)
        do_chunk(c * 2 + 1, 1)
        return carry

    lax.fori_loop(0, NCH // 2, chunk, 0)
    do_chunk(NCH - 1, 0)
    out_copy(NCH - 2, 1).wait()
    out_copy(NCH - 1, 0).wait()


def _body(idx_hbm, table_hbm, out_hbm, idx_v, gbuf, obuf, osems, *sems):
    wid = lax.axis_index("s") * NC + lax.axis_index("c")
    out_base = wid * ROWS_PER_W

    pltpu.sync_copy(idx_hbm.at[pl.ds(wid * NG, NG)], idx_v)

    def gather(j, b):
        return pltpu.make_async_copy(
            table_hbm.at[idx_v.at[j]], gbuf.at[b], sems[b])

    def out_copy(j, ob):
        return pltpu.make_async_copy(
            obuf.at[ob], out_hbm.at[pl.ds(out_base + j * RG, RG)], osems[ob])

    for b in range(NB):
        gather(b, b).start()

    def group(g, carry):
        for b in range(NB):
            j = g * NB + b
            ob = b % 2
            gather(j, b).wait()

            @pl.when(j >= 2)
            def _():
                out_copy(j - 2, ob).wait()

            def row(r, carry):
                base = r * S

                def bf(s, h):
                    return plsc.bitcast(
                        gbuf[b, base + s, pl.ds(h * 16, 16)], jnp.bfloat16)

                for h in range(2):
                    t0 = bf(0, h) + bf(1, h)
                    t1 = bf(2, h) + bf(3, h)
                    t2 = bf(4, h) + bf(5, h)
                    t3 = bf(6, h) + bf(7, h)
                    u = plsc.bitcast((t0 + t1) + (t2 + t3), jnp.uint32)
                    lo = plsc.bitcast(u << 16, jnp.float32)
                    hi = plsc.bitcast(u & _HI, jnp.float32)
                    obuf[ob, r, pl.ds(h * 32, 16)] = lo * (1.0 / S)
                    obuf[ob, r, pl.ds(h * 32 + 16, 16)] = hi * (1.0 / S)
                return carry

            lax.fori_loop(0, RG, row, 0, unroll=4)
            out_copy(j, ob).start()

            @pl.when(j + NB < NG)
            def _():
                gather(j + NB, b).start()
        return carry

    lax.fori_loop(0, NG // NB, group, 0)
    out_copy(NG - 2, 0).wait()
    out_copy(NG - 1, 1).wait()


@jax.jit
def _run(idx3, table):
    idx = idx3.reshape(NW * NG, G)
    tf32 = table.reshape(F32_ROWS, 128)
    mesh = plsc.VectorSubcoreMesh(
        core_axis_name="c", subcore_axis_name="s",
        num_cores=NC, num_subcores=NS)
    conv = pl.kernel(
        _conv_body,
        out_type=jax.ShapeDtypeStruct((PK_ROWS, 128), jnp.uint32),
        mesh=mesh,
        scratch_types=[
            pltpu.VMEM((2, CH, 128), jnp.float32),
            pltpu.VMEM((2, CH // 2, 128), jnp.uint32),
            [pltpu.SemaphoreType.DMA] * 2,
            [pltpu.SemaphoreType.DMA] * 2,
        ],
        compiler_params=pltpu.CompilerParams(
            use_tc_tiling_on_sc=False, needs_layout_passes=False),
    )
    table_bf = conv(tf32).reshape(N_ITEMS, HU)
    kfn = pl.kernel(
        _body,
        out_type=jax.ShapeDtypeStruct((B * L, HIDDEN), jnp.float32),
        mesh=mesh,
        scratch_types=[
            pltpu.VMEM((NG, G), jnp.int32),
            pltpu.VMEM((NB, G, HU), jnp.uint32),
            pltpu.VMEM((2, RG, HIDDEN), jnp.float32),
            [pltpu.SemaphoreType.DMA] * 2,
        ] + [pltpu.SemaphoreType.DMA] * NB,
        compiler_params=pltpu.CompilerParams(
            use_tc_tiling_on_sc=False, needs_layout_passes=False),
    )
    return kfn(idx, table_bf).reshape(B, L, HIDDEN)


def kernel(batch_basket, item_embedding):
    return _run(batch_basket, item_embedding)


# speedup vs baseline: 1.0021x; 1.0002x over previous
import jax
import jax.numpy as jnp
from jax import lax
from jax.experimental import pallas as pl
from jax.experimental.pallas import tpu as pltpu
from jax.experimental.pallas import tpu_sc as plsc

import numpy as np

B, L, S = 1024, 50, 8
HIDDEN = 64
N_ITEMS = 100000

NC, NS = 2, 16
NW = NC * NS
IDX_PER_W = (B * L * S) // NW
G = 128
RG = G // S
NG = IDX_PER_W // G
ROWS_PER_W = IDX_PER_W // S
NB = 10
HU = HIDDEN // 2

F32_ROWS = N_ITEMS * HIDDEN // 128
CH = 128
BASE_RPW = F32_ROWS // NW
EXTRA_W = (F32_ROWS - BASE_RPW * NW) // 2
NCH = 13
PK_ROWS = F32_ROWS // 2

_RND = np.uint32(0x8000)
_HI = np.uint32(0xFFFF0000)


def _conv_body(tf32_hbm, tb_hbm, ibuf, obuf, isems, osems):
    wid = lax.axis_index("s") * NC + lax.axis_index("c")
    r0 = wid * BASE_RPW + 2 * jnp.minimum(wid, EXTRA_W)
    n = BASE_RPW + jnp.where(wid < EXTRA_W, 2, 0)

    def start_of(c):
        return r0 + jnp.minimum(c * CH, n - CH)

    def in_copy(c, bb):
        return pltpu.make_async_copy(
            tf32_hbm.at[pl.ds(start_of(c), CH)], ibuf.at[bb], isems[bb])

    def out_copy(c, bb):
        return pltpu.make_async_copy(
            obuf.at[bb], tb_hbm.at[pl.ds(start_of(c) // 2, CH // 2)],
            osems[bb])

    in_copy(0, 0).start()
    in_copy(1, 1).start()

    def do_chunk(cc, bb):
        in_copy(cc, bb).wait()

        @pl.when(cc >= 2)
        def _():
            out_copy(cc - 2, bb).wait()

        def row(rp, carry):
            for parity in range(2):
                rr = 2 * rp + parity
                for g in range(4):
                    ua = plsc.bitcast(ibuf[bb, rr, pl.ds(g * 32, 16)],
                                      jnp.uint32)
                    ub = plsc.bitcast(ibuf[bb, rr, pl.ds(g * 32 + 16, 16)],
                                      jnp.uint32)
                    w = ((ua + _RND) >> 16) | ((ub + _RND) & _HI)
                    obuf[bb, rp, pl.ds(parity * 64 + g * 16, 16)] = w
            return carry

        lax.fori_loop(0, CH // 2, row, 0, unroll=2)
        out_copy(cc, bb).start()

        @pl.when(cc + 2 < NCH)
        def _():
            in_copy(cc + 2, bb).start()

    def chunk(c, carry):
        do_chunk(c * 2, 0)
        do_chunk(c * 2 + 1, 1)
        return carry

    lax.fori_loop(0, NCH // 2, chunk, 0)
    do_chunk(NCH - 1, 0)
    out_copy(NCH - 2, 1).wait()
    out_copy(NCH - 1, 0).wait()


def _body(idx_hbm, table_hbm, out_hbm, idx_v, gbuf, obuf, osems, *sems):
    wid = lax.axis_index("s") * NC + lax.axis_index("c")
    out_base = wid * ROWS_PER_W

    pltpu.sync_copy(idx_hbm.at[pl.ds(wid * NG, NG)], idx_v)

    def gather(j, b):
        return pltpu.make_async_copy(
            table_hbm.at[idx_v.at[j]], gbuf.at[b], sems[b])

    def out_copy(j, ob):
        return pltpu.make_async_copy(
            obuf.at[ob], out_hbm.at[pl.ds(out_base + j * RG, RG)], osems[ob])

    for b in range(NB):
        gather(b, b).start()

    def group(g, carry):
        for b in range(NB):
            j = g * NB + b
            ob = b % 2
            gather(j, b).wait()

            @pl.when(j >= 2)
            def _():
                out_copy(j - 2, ob).wait()

            def row(r, carry):
                base = r * S

                def bf(s, h):
                    return plsc.bitcast(
                        gbuf[b, base + s, pl.ds(h * 16, 16)], jnp.bfloat16)

                for h in range(2):
                    t0 = bf(0, h) + bf(1, h)
                    t1 = bf(2, h) + bf(3, h)
                    t2 = bf(4, h) + bf(5, h)
                    t3 = bf(6, h) + bf(7, h)
                    u = plsc.bitcast((t0 + t1) + (t2 + t3), jnp.uint32)
                    lo = plsc.bitcast(u << 16, jnp.float32)
                    hi = plsc.bitcast(u & _HI, jnp.float32)
                    obuf[ob, r, pl.ds(h * 32, 16)] = lo * (1.0 / S)
                    obuf[ob, r, pl.ds(h * 32 + 16, 16)] = hi * (1.0 / S)
                return carry

            lax.fori_loop(0, RG, row, 0, unroll=4)
            out_copy(j, ob).start()

            @pl.when(j + NB < NG)
            def _():
                gather(j + NB, b).start()
        return carry

    lax.fori_loop(0, NG // NB, group, 0)
    out_copy(NG - 2, 0).wait()
    out_copy(NG - 1, 1).wait()


@jax.jit
def _run(idx3, table):
    tf32 = table.reshape(F32_ROWS, 128)
    mesh = plsc.VectorSubcoreMesh(
        core_axis_name="c", subcore_axis_name="s",
        num_cores=NC, num_subcores=NS)
    conv = pl.kernel(
        _conv_body,
        out_type=jax.ShapeDtypeStruct((PK_ROWS, 128), jnp.uint32),
        mesh=mesh,
        scratch_types=[
            pltpu.VMEM((2, CH, 128), jnp.float32),
            pltpu.VMEM((2, CH // 2, 128), jnp.uint32),
            [pltpu.SemaphoreType.DMA] * 2,
            [pltpu.SemaphoreType.DMA] * 2,
        ],
        compiler_params=pltpu.CompilerParams(
            use_tc_tiling_on_sc=False, needs_layout_passes=False),
    )
    table_bf = conv(tf32).reshape(N_ITEMS, HU)
    idx = idx3.reshape(NW * NG, G)
    kfn = pl.kernel(
        _body,
        out_type=jax.ShapeDtypeStruct((B * L, HIDDEN), jnp.float32),
        mesh=mesh,
        scratch_types=[
            pltpu.VMEM((NG, G), jnp.int32),
            pltpu.VMEM((NB, G, HU), jnp.uint32),
            pltpu.VMEM((2, RG, HIDDEN), jnp.float32),
            [pltpu.SemaphoreType.DMA] * 2,
        ] + [pltpu.SemaphoreType.DMA] * NB,
        compiler_params=pltpu.CompilerParams(
            use_tc_tiling_on_sc=False, needs_layout_passes=False),
    )
    return kfn(idx, table_bf).reshape(B, L, HIDDEN)


def kernel(batch_basket, item_embedding):
    return _run(batch_basket, item_embedding)
